# Initial kernel scaffold; baseline (speedup 1.0000x reference)
import jax
import jax.numpy as jnp
from jax.experimental import pallas as pl


def kernel(query, key, value):
    raise NotImplementedError("write your pallas kernel here")


# speedup vs baseline: 3.6116x; 3.6116x over previous
import functools
import math

import jax
import jax.numpy as jnp
from jax import lax
from jax.experimental import pallas as pl
from jax.experimental.pallas import tpu as pltpu
from jax.experimental.pallas import tpu_sc as plsc

B, H, Q, S, D = 8, 16, 8, 4096, 128
K_KEEP = 2048
SINK = 4
K_CAND = K_KEEP - SINK


def _attn_body(q_ref, k_ref, v_ref, o_ref, cls_ref, aux_ref, acc_ref):
    h = pl.program_id(1)
    q = q_ref[0, 0]
    k = k_ref[0, 0]
    v = v_ref[0, 0]
    scale = 1.0 / math.sqrt(D)
    s = jnp.dot(q, k.T, preferred_element_type=jnp.float32) * scale
    m = jnp.max(s, axis=-1, keepdims=True)
    p = jnp.exp(s - m)
    l = jnp.sum(p, axis=-1, keepdims=True)
    w = p / l
    o_ref[0, 0] = jnp.dot(w, v, preferred_element_type=jnp.float32)
    wsum = jnp.sum(w, axis=0, keepdims=True)

    @pl.when(h == 0)
    def _():
        acc_ref[...] = wsum

    @pl.when(h != 0)
    def _():
        acc_ref[...] = acc_ref[...] + wsum

    @pl.when(h == H - 1)
    def _():
        hv = acc_ref[...]
        bits = lax.bitcast_convert_type(hv, jnp.int32)
        pos = lax.broadcasted_iota(jnp.int32, (1, S), 1)
        iscand = pos >= SINK

        def bs_body(_, lohi):
            lo, hi = lohi
            mid = lo + (hi - lo + 1) // 2
            cnt = jnp.sum(jnp.where(iscand & (bits >= mid), 1, 0))
            take = cnt >= K_CAND
            return (jnp.where(take, mid, lo), jnp.where(take, hi, mid - 1))

        lo, _ = lax.fori_loop(
            0, 31, bs_body, (jnp.int32(0), jnp.int32(2**31 - 2)))
        t = lo
        gt = iscand & (bits > t)
        eq = iscand & (bits == t)
        g = jnp.sum(jnp.where(gt, 1, 0))
        r = K_CAND - g
        cls = jnp.where(pos < SINK, 2, jnp.where(gt, 2, jnp.where(eq, 1, 0)))
        cls_ref[0] = cls
        aux_ref[0] = jnp.full((1, 128), r, jnp.int32)


def _tc_attention(query, key, value):
    grid = (B, H)
    out = pl.pallas_call(
        _attn_body,
        grid=grid,
        in_specs=[
            pl.BlockSpec((1, 1, Q, D), lambda b, h: (b, h, 0, 0)),
            pl.BlockSpec((1, 1, S, D), lambda b, h: (b, h, 0, 0)),
            pl.BlockSpec((1, 1, S, D), lambda b, h: (b, h, 0, 0)),
        ],
        out_specs=[
            pl.BlockSpec((1, 1, Q, D), lambda b, h: (b, h, 0, 0)),
            pl.BlockSpec((1, 1, S), lambda b, h: (b, 0, 0)),
            pl.BlockSpec((1, 1, 128), lambda b, h: (b, 0, 0)),
        ],
        out_shape=[
            jax.ShapeDtypeStruct((B, H, Q, D), jnp.float32),
            jax.ShapeDtypeStruct((B, 1, S), jnp.int32),
            jax.ShapeDtypeStruct((B, 1, 128), jnp.int32),
        ],
        scratch_shapes=[pltpu.VMEM((1, S), jnp.float32)],
    )(query, key, value)
    return out


NC, NS, L = 2, 16, 16
NW = NC * NS
PAIRS = (B * H) // NW
CHUNK = 128
NCHUNK = K_KEEP // CHUNK


def _sc_evict(key_flat, value_flat, cls, aux):
    mesh = plsc.VectorSubcoreMesh(core_axis_name="c", subcore_axis_name="s")

    @functools.partial(
        pl.kernel,
        mesh=mesh,
        compiler_params=pltpu.CompilerParams(needs_layout_passes=False),
        out_type=[
            jax.ShapeDtypeStruct((B * H * K_KEEP, D), jnp.float32),
            jax.ShapeDtypeStruct((B * H * K_KEEP, D), jnp.float32),
        ],
        scratch_types=[
            pltpu.VMEM((S,), jnp.int32),
            pltpu.VMEM((128,), jnp.int32),
            pltpu.VMEM((K_KEEP + L,), jnp.int32),
            pltpu.VMEM((NCHUNK, CHUNK), jnp.int32),
            pltpu.VMEM((CHUNK, D), jnp.float32),
            pltpu.VMEM((CHUNK, D), jnp.float32),
            pltpu.SemaphoreType.DMA,
            pltpu.SemaphoreType.DMA,
        ],
    )
    def body(key_hbm, value_hbm, cls_hbm, aux_hbm, outk_hbm, outv_hbm,
             cls_v, aux_v, idx_v, rows_v, bk, bv, gsk, gsv):
        cid = lax.axis_index("c")
        sid = lax.axis_index("s")
        wid = sid * NC + cid
        b = wid % B
        hgrp = wid // B

        pltpu.sync_copy(cls_hbm.at[pl.ds(pl.multiple_of(b * S, S), S)], cls_v)
        pltpu.sync_copy(aux_hbm.at[pl.ds(pl.multiple_of(b * 128, 128), 128)],
                        aux_v)
        r = aux_v[pl.ds(0, L)][0]

        def comp_body(i, carry):
            nw_, nt_ = carry
            v = cls_v[pl.ds(i * L, L)]
            posv = i * L + lax.iota(jnp.int32, L)
            is2 = v == 2
            is1 = v == 1
            tp = plsc.cumsum(jnp.where(is1, 1, 0))
            take1 = is1 & ((nt_ + tp) <= r)
            keep = jnp.logical_or(is2, take1)
            plsc.store_compressed(idx_v.at[pl.ds(nw_, L)], posv, mask=keep)
            nk = plsc.all_reduce_population_count(keep)[0]
            ntk = plsc.all_reduce_population_count(take1)[0]
            return (nw_ + nk, nt_ + ntk)

        lax.fori_loop(0, S // L, comp_body, (jnp.int32(0), jnp.int32(0)))

        def pair_body(j, _):
            h = hgrp * PAIRS + j
            tbl_off = (b * H + h) * S
            out_base = (b * H + h) * K_KEEP

            def rows_body(i, _):
                c = i // (CHUNK // L)
                o = (i % (CHUNK // L)) * L
                rows_v[c, pl.ds(o, L)] = idx_v[pl.ds(i * L, L)] + tbl_off
                return 0

            lax.fori_loop(0, K_KEEP // L, rows_body, 0)

            def chunk_body(c, _):
                ck = pltpu.async_copy(key_hbm.at[rows_v.at[c]], bk, gsk)
                cv = pltpu.async_copy(value_hbm.at[rows_v.at[c]], bv, gsv)
                ck.wait()
                cv.wait()
                dst = pl.ds(pl.multiple_of(out_base + c * CHUNK, CHUNK), CHUNK)
                pltpu.sync_copy(bk, outk_hbm.at[dst])
                pltpu.sync_copy(bv, outv_hbm.at[dst])
                return 0

            lax.fori_loop(0, NCHUNK, chunk_body, 0)
            return 0

        lax.fori_loop(0, PAIRS, pair_body, 0)

    return body(key_flat, value_flat, cls, aux)


def kernel(query, key, value):
    attn_out, cls, aux = _tc_attention(query, key, value)
    key_flat = key.reshape(B * H * S, D)
    value_flat = value.reshape(B * H * S, D)
    outk, outv = _sc_evict(key_flat, value_flat,
                           cls.reshape(B * S), aux.reshape(B * 128))
    new_key = outk.reshape(B, H, K_KEEP, D)
    new_value = outv.reshape(B, H, K_KEEP, D)
    return attn_out, (new_key, new_value)


# speedup vs baseline: 3.8403x; 1.0633x over previous
import functools
import math

import jax
import jax.numpy as jnp
from jax import lax
from jax.experimental import pallas as pl
from jax.experimental.pallas import tpu as pltpu
from jax.experimental.pallas import tpu_sc as plsc

B, H, Q, S, D = 8, 16, 8, 4096, 128
K_KEEP = 2048
SINK = 4
K_CAND = K_KEEP - SINK


def _attn_body(q_ref, k_ref, v_ref, o_ref, cls_ref, aux_ref, acc_ref):
    h = pl.program_id(1)
    q = q_ref[0, 0]
    k = k_ref[0, 0]
    v = v_ref[0, 0]
    scale = 1.0 / math.sqrt(D)
    s = jnp.dot(q, k.T, preferred_element_type=jnp.float32) * scale
    m = jnp.max(s, axis=-1, keepdims=True)
    p = jnp.exp(s - m)
    l = jnp.sum(p, axis=-1, keepdims=True)
    w = p / l
    o_ref[0, 0] = jnp.dot(w, v, preferred_element_type=jnp.float32)
    wsum = jnp.sum(w, axis=0, keepdims=True)

    @pl.when(h == 0)
    def _():
        acc_ref[...] = wsum

    @pl.when(h != 0)
    def _():
        acc_ref[...] = acc_ref[...] + wsum

    @pl.when(h == H - 1)
    def _():
        hv = acc_ref[...]
        bits = lax.bitcast_convert_type(hv, jnp.int32)
        pos = lax.broadcasted_iota(jnp.int32, (1, S), 1)
        iscand = pos >= SINK

        def bs_body(_, lohi):
            lo, hi = lohi
            mid = lo + (hi - lo + 1) // 2
            cnt = jnp.sum(jnp.where(iscand & (bits >= mid), 1, 0))
            take = cnt >= K_CAND
            return (jnp.where(take, mid, lo), jnp.where(take, hi, mid - 1))

        lo, _ = lax.fori_loop(
            0, 31, bs_body, (jnp.int32(0), jnp.int32(2**31 - 2)))
        t = lo
        gt = iscand & (bits > t)
        eq = iscand & (bits == t)
        g = jnp.sum(jnp.where(gt, 1, 0))
        r = K_CAND - g
        cls = jnp.where(pos < SINK, 2, jnp.where(gt, 2, jnp.where(eq, 1, 0)))
        cls_ref[0] = cls
        aux_ref[0] = jnp.full((1, 128), r, jnp.int32)


def _tc_attention(query, key, value):
    grid = (B, H)
    out = pl.pallas_call(
        _attn_body,
        grid=grid,
        in_specs=[
            pl.BlockSpec((1, 1, Q, D), lambda b, h: (b, h, 0, 0)),
            pl.BlockSpec((1, 1, S, D), lambda b, h: (b, h, 0, 0)),
            pl.BlockSpec((1, 1, S, D), lambda b, h: (b, h, 0, 0)),
        ],
        out_specs=[
            pl.BlockSpec((1, 1, Q, D), lambda b, h: (b, h, 0, 0)),
            pl.BlockSpec((1, 1, S), lambda b, h: (b, 0, 0)),
            pl.BlockSpec((1, 1, 128), lambda b, h: (b, 0, 0)),
        ],
        out_shape=[
            jax.ShapeDtypeStruct((B, H, Q, D), jnp.float32),
            jax.ShapeDtypeStruct((B, 1, S), jnp.int32),
            jax.ShapeDtypeStruct((B, 1, 128), jnp.int32),
        ],
        scratch_shapes=[pltpu.VMEM((1, S), jnp.float32)],
    )(query, key, value)
    return out


NC, NS, L = 2, 16, 16
NW = NC * NS
PAIRS = (B * H) // NW
CHUNK = 128
NCHUNK = K_KEEP // CHUNK


def _sc_evict(key_flat, value_flat, cls, aux):
    mesh = plsc.VectorSubcoreMesh(core_axis_name="c", subcore_axis_name="s")

    @functools.partial(
        pl.kernel,
        mesh=mesh,
        compiler_params=pltpu.CompilerParams(needs_layout_passes=False),
        out_type=[
            jax.ShapeDtypeStruct((B * H * K_KEEP, D), jnp.float32),
            jax.ShapeDtypeStruct((B * H * K_KEEP, D), jnp.float32),
        ],
        scratch_types=[
            pltpu.VMEM((S,), jnp.int32),
            pltpu.VMEM((128,), jnp.int32),
            pltpu.VMEM((K_KEEP + L,), jnp.int32),
            pltpu.VMEM((NCHUNK, CHUNK), jnp.int32),
            pltpu.VMEM((CHUNK, D), jnp.float32),
            pltpu.VMEM((CHUNK, D), jnp.float32),
            pltpu.VMEM((CHUNK, D), jnp.float32),
            pltpu.VMEM((CHUNK, D), jnp.float32),
            pltpu.SemaphoreType.DMA,
            pltpu.SemaphoreType.DMA,
            pltpu.SemaphoreType.DMA,
            pltpu.SemaphoreType.DMA,
            pltpu.SemaphoreType.DMA,
            pltpu.SemaphoreType.DMA,
            pltpu.SemaphoreType.DMA,
            pltpu.SemaphoreType.DMA,
        ],
    )
    def body(key_hbm, value_hbm, cls_hbm, aux_hbm, outk_hbm, outv_hbm,
             cls_v, aux_v, idx_v, rows_v, bk0, bk1, bv0, bv1,
             gsk0, gsk1, gsv0, gsv1, wsk0, wsk1, wsv0, wsv1):
        cid = lax.axis_index("c")
        sid = lax.axis_index("s")
        wid = sid * NC + cid
        b = wid % B
        hgrp = wid // B

        pltpu.sync_copy(cls_hbm.at[pl.ds(pl.multiple_of(b * S, S), S)], cls_v)
        pltpu.sync_copy(aux_hbm.at[pl.ds(pl.multiple_of(b * 128, 128), 128)],
                        aux_v)
        r = aux_v[pl.ds(0, L)][0]

        def comp_body(i, carry):
            nw_, nt_ = carry
            v = cls_v[pl.ds(i * L, L)]
            posv = i * L + lax.iota(jnp.int32, L)
            is2 = v == 2
            is1 = v == 1
            tp = plsc.cumsum(jnp.where(is1, 1, 0))
            take1 = is1 & ((nt_ + tp) <= r)
            keep = jnp.logical_or(is2, take1)
            plsc.store_compressed(idx_v.at[pl.ds(nw_, L)], posv, mask=keep)
            nk = plsc.all_reduce_population_count(keep)[0]
            ntk = plsc.all_reduce_population_count(take1)[0]
            return (nw_ + nk, nt_ + ntk)

        lax.fori_loop(0, S // L, comp_body, (jnp.int32(0), jnp.int32(0)))

        def pair_body(j, _):
            h = hgrp * PAIRS + j
            tbl_off = (b * H + h) * S
            out_base = (b * H + h) * K_KEEP

            def rows_body(i, _):
                c = i // (CHUNK // L)
                o = (i % (CHUNK // L)) * L
                rows_v[c, pl.ds(o, L)] = idx_v[pl.ds(i * L, L)] + tbl_off
                return 0

            lax.fori_loop(0, K_KEEP // L, rows_body, 0)

            bufs = ((bk0, bv0, gsk0, gsv0, wsk0, wsv0),
                    (bk1, bv1, gsk1, gsv1, wsk1, wsv1))

            def _dst(c):
                return pl.ds(pl.multiple_of(out_base + c * CHUNK, CHUNK),
                             CHUNK)

            def _gather(c, bk, bv, gsk, gsv):
                pltpu.async_copy(key_hbm.at[rows_v.at[c]], bk, gsk)
                pltpu.async_copy(value_hbm.at[rows_v.at[c]], bv, gsv)

            def _wait(src, dstref, sem):
                pltpu.make_async_copy(src, dstref, sem).wait()

            def step(i, bk, bv, gsk, gsv, wsk, wsv,
                     bkq, bvq, gskq, gsvq, wskq, wsvq):
                @pl.when(i < NCHUNK)
                def _():
                    @pl.when(i >= 2)
                    def _():
                        _wait(bk, outk_hbm.at[_dst(i - 2)], wsk)
                        _wait(bv, outv_hbm.at[_dst(i - 2)], wsv)
                    _gather(i, bk, bv, gsk, gsv)

                j = i - 1

                @pl.when(jnp.logical_and(j >= 0, j < NCHUNK))
                def _():
                    _wait(key_hbm.at[pl.ds(0, CHUNK)], bkq, gskq)
                    _wait(value_hbm.at[pl.ds(0, CHUNK)], bvq, gsvq)
                    pltpu.async_copy(bkq, outk_hbm.at[_dst(j)], wskq)
                    pltpu.async_copy(bvq, outv_hbm.at[_dst(j)], wsvq)

            def pipe_body(g, _):
                i0 = g * 2
                step(i0, *bufs[0], *bufs[1])
                step(i0 + 1, *bufs[1], *bufs[0])
                return 0

            lax.fori_loop(0, (NCHUNK + 2 + 1) // 2, pipe_body, 0)
            _wait(bk0, outk_hbm.at[_dst(NCHUNK - 2)], wsk0)
            _wait(bv0, outv_hbm.at[_dst(NCHUNK - 2)], wsv0)
            _wait(bk1, outk_hbm.at[_dst(NCHUNK - 1)], wsk1)
            _wait(bv1, outv_hbm.at[_dst(NCHUNK - 1)], wsv1)
            return 0

        lax.fori_loop(0, PAIRS, pair_body, 0)

    return body(key_flat, value_flat, cls, aux)


def kernel(query, key, value):
    attn_out, cls, aux = _tc_attention(query, key, value)
    key_flat = key.reshape(B * H * S, D)
    value_flat = value.reshape(B * H * S, D)
    outk, outv = _sc_evict(key_flat, value_flat,
                           cls.reshape(B * S), aux.reshape(B * 128))
    new_key = outk.reshape(B, H, K_KEEP, D)
    new_value = outv.reshape(B, H, K_KEEP, D)
    return attn_out, (new_key, new_value)


# speedup vs baseline: 4.1519x; 1.0811x over previous
import functools
import math

import jax
import jax.numpy as jnp
from jax import lax
from jax.experimental import pallas as pl
from jax.experimental.pallas import tpu as pltpu
from jax.experimental.pallas import tpu_sc as plsc

B, H, Q, S, D = 8, 16, 8, 4096, 128
K_KEEP = 2048
SINK = 4
K_CAND = K_KEEP - SINK


def _attn_body(q_ref, k_ref, v_ref, o_ref, hs_ref, acc_ref):
    h = pl.program_id(1)
    q = q_ref[0, 0]
    k = k_ref[0, 0]
    v = v_ref[0, 0]
    scale = 1.0 / math.sqrt(D)
    s = jnp.dot(q, k.T, preferred_element_type=jnp.float32) * scale
    m = jnp.max(s, axis=-1, keepdims=True)
    p = jnp.exp(s - m)
    l = jnp.sum(p, axis=-1, keepdims=True)
    rl = 1.0 / l
    o_ref[0, 0] = jnp.dot(p, v, preferred_element_type=jnp.float32) * rl
    wsum = jnp.dot(rl.T, p, preferred_element_type=jnp.float32)

    @pl.when(h == 0)
    def _():
        acc_ref[...] = wsum

    @pl.when(h != 0)
    def _():
        acc_ref[...] = acc_ref[...] + wsum

    @pl.when(h == H - 1)
    def _():
        hs_ref[0] = acc_ref[...]


def _tc_attention(query, key, value):
    grid = (B, H)
    out = pl.pallas_call(
        _attn_body,
        grid=grid,
        in_specs=[
            pl.BlockSpec((1, 1, Q, D), lambda b, h: (b, h, 0, 0)),
            pl.BlockSpec((1, 1, S, D), lambda b, h: (b, h, 0, 0)),
            pl.BlockSpec((1, 1, S, D), lambda b, h: (b, h, 0, 0)),
        ],
        out_specs=[
            pl.BlockSpec((1, 1, Q, D), lambda b, h: (b, h, 0, 0)),
            pl.BlockSpec((1, 1, S), lambda b, h: (b, 0, 0)),
        ],
        out_shape=[
            jax.ShapeDtypeStruct((B, H, Q, D), jnp.float32),
            jax.ShapeDtypeStruct((B, 1, S), jnp.float32),
        ],
        scratch_shapes=[pltpu.VMEM((1, S), jnp.float32)],
    )(query, key, value)
    return out


def _select_body(hs_ref, cls_ref, aux_ref):
    hv = hs_ref[...]
    bits = lax.bitcast_convert_type(hv, jnp.int32)
    pos = (lax.broadcasted_iota(jnp.int32, hv.shape, 1) * 128
           + lax.broadcasted_iota(jnp.int32, hv.shape, 2))
    iscand = pos >= SINK

    def bs_body(_, lohi):
        lo, hi = lohi
        mid = lo + (hi - lo + 1) // 2
        cnt = jnp.sum(jnp.where(iscand & (bits >= mid), 1, 0),
                      axis=(1, 2), keepdims=True)
        take = cnt >= K_CAND
        return (jnp.where(take, mid, lo), jnp.where(take, hi, mid - 1))

    init = (jnp.zeros((B, 1, 1), jnp.int32),
            jnp.full((B, 1, 1), 2**31 - 2, jnp.int32))
    lo, _ = lax.fori_loop(0, 31, bs_body, init)
    t = lo
    gt = iscand & (bits > t)
    eq = iscand & (bits == t)
    g = jnp.sum(jnp.where(gt, 1, 0), axis=(1, 2), keepdims=True)
    r = K_CAND - g
    cls_ref[...] = jnp.where(pos < SINK, 2,
                             jnp.where(gt, 2, jnp.where(eq, 1, 0)))
    aux_ref[...] = jnp.broadcast_to(r, (B, 1, 128)).astype(jnp.int32)


def _tc_select(head_sum):
    SCH = S // 128
    out = pl.pallas_call(
        _select_body,
        out_shape=[
            jax.ShapeDtypeStruct((B, SCH, 128), jnp.int32),
            jax.ShapeDtypeStruct((B, 1, 128), jnp.int32),
        ],
    )(head_sum.reshape(B, SCH, 128))
    return out


NC, NS, L = 2, 16, 16
NW = NC * NS
PAIRS = (B * H) // NW
CHUNK = 128
NCHUNK = K_KEEP // CHUNK


def _sc_evict(key_flat, value_flat, cls, aux):
    mesh = plsc.VectorSubcoreMesh(core_axis_name="c", subcore_axis_name="s")

    @functools.partial(
        pl.kernel,
        mesh=mesh,
        compiler_params=pltpu.CompilerParams(needs_layout_passes=False),
        out_type=[
            jax.ShapeDtypeStruct((B * H * K_KEEP, D), jnp.float32),
            jax.ShapeDtypeStruct((B * H * K_KEEP, D), jnp.float32),
        ],
        scratch_types=[
            pltpu.VMEM((S,), jnp.int32),
            pltpu.VMEM((128,), jnp.int32),
            pltpu.VMEM((K_KEEP + L,), jnp.int32),
            pltpu.VMEM((NCHUNK, CHUNK), jnp.int32),
            pltpu.VMEM((CHUNK, D), jnp.float32),
            pltpu.VMEM((CHUNK, D), jnp.float32),
            pltpu.VMEM((CHUNK, D), jnp.float32),
            pltpu.VMEM((CHUNK, D), jnp.float32),
            pltpu.SemaphoreType.DMA,
            pltpu.SemaphoreType.DMA,
            pltpu.SemaphoreType.DMA,
            pltpu.SemaphoreType.DMA,
            pltpu.SemaphoreType.DMA,
            pltpu.SemaphoreType.DMA,
            pltpu.SemaphoreType.DMA,
            pltpu.SemaphoreType.DMA,
        ],
    )
    def body(key_hbm, value_hbm, cls_hbm, aux_hbm, outk_hbm, outv_hbm,
             cls_v, aux_v, idx_v, rows_v, bk0, bk1, bv0, bv1,
             gsk0, gsk1, gsv0, gsv1, wsk0, wsk1, wsv0, wsv1):
        cid = lax.axis_index("c")
        sid = lax.axis_index("s")
        wid = sid * NC + cid
        b = wid % B
        hgrp = wid // B

        pltpu.sync_copy(cls_hbm.at[pl.ds(pl.multiple_of(b * S, S), S)], cls_v)
        pltpu.sync_copy(aux_hbm.at[pl.ds(pl.multiple_of(b * 128, 128), 128)],
                        aux_v)
        r = aux_v[pl.ds(0, L)][0]

        def comp_body(i, carry):
            nw_, nt_ = carry
            v = cls_v[pl.ds(i * L, L)]
            posv = i * L + lax.iota(jnp.int32, L)
            is2 = v == 2
            is1 = v == 1
            tp = plsc.cumsum(jnp.where(is1, 1, 0))
            take1 = is1 & ((nt_ + tp) <= r)
            keep = jnp.logical_or(is2, take1)
            plsc.store_compressed(idx_v.at[pl.ds(nw_, L)], posv, mask=keep)
            nk = plsc.all_reduce_population_count(keep)[0]
            ntk = plsc.all_reduce_population_count(take1)[0]
            return (nw_ + nk, nt_ + ntk)

        lax.fori_loop(0, S // L, comp_body, (jnp.int32(0), jnp.int32(0)))

        def pair_body(j, _):
            h = hgrp * PAIRS + j
            tbl_off = (b * H + h) * S
            out_base = (b * H + h) * K_KEEP

            def rows_body(i, _):
                c = i // (CHUNK // L)
                o = (i % (CHUNK // L)) * L
                rows_v[c, pl.ds(o, L)] = idx_v[pl.ds(i * L, L)] + tbl_off
                return 0

            lax.fori_loop(0, K_KEEP // L, rows_body, 0)

            bufs = ((bk0, bv0, gsk0, gsv0, wsk0, wsv0),
                    (bk1, bv1, gsk1, gsv1, wsk1, wsv1))

            def _dst(c):
                return pl.ds(pl.multiple_of(out_base + c * CHUNK, CHUNK),
                             CHUNK)

            def _gather(c, bk, bv, gsk, gsv):
                pltpu.async_copy(key_hbm.at[rows_v.at[c]], bk, gsk)
                pltpu.async_copy(value_hbm.at[rows_v.at[c]], bv, gsv)

            def _wait(src, dstref, sem):
                pltpu.make_async_copy(src, dstref, sem).wait()

            def step(i, bk, bv, gsk, gsv, wsk, wsv,
                     bkq, bvq, gskq, gsvq, wskq, wsvq):
                @pl.when(i < NCHUNK)
                def _():
                    @pl.when(i >= 2)
                    def _():
                        _wait(bk, outk_hbm.at[_dst(i - 2)], wsk)
                        _wait(bv, outv_hbm.at[_dst(i - 2)], wsv)
                    _gather(i, bk, bv, gsk, gsv)

                j = i - 1

                @pl.when(jnp.logical_and(j >= 0, j < NCHUNK))
                def _():
                    _wait(key_hbm.at[pl.ds(0, CHUNK)], bkq, gskq)
                    _wait(value_hbm.at[pl.ds(0, CHUNK)], bvq, gsvq)
                    pltpu.async_copy(bkq, outk_hbm.at[_dst(j)], wskq)
                    pltpu.async_copy(bvq, outv_hbm.at[_dst(j)], wsvq)

            def pipe_body(g, _):
                i0 = g * 2
                step(i0, *bufs[0], *bufs[1])
                step(i0 + 1, *bufs[1], *bufs[0])
                return 0

            lax.fori_loop(0, (NCHUNK + 2 + 1) // 2, pipe_body, 0)
            _wait(bk0, outk_hbm.at[_dst(NCHUNK - 2)], wsk0)
            _wait(bv0, outv_hbm.at[_dst(NCHUNK - 2)], wsv0)
            _wait(bk1, outk_hbm.at[_dst(NCHUNK - 1)], wsk1)
            _wait(bv1, outv_hbm.at[_dst(NCHUNK - 1)], wsv1)
            return 0

        lax.fori_loop(0, PAIRS, pair_body, 0)

    return body(key_flat, value_flat, cls, aux)


def kernel(query, key, value):
    attn_out, head_sum = _tc_attention(query, key, value)
    cls, aux = _tc_select(head_sum)
    key_flat = key.reshape(B * H * S, D)
    value_flat = value.reshape(B * H * S, D)
    outk, outv = _sc_evict(key_flat, value_flat,
                           cls.reshape(B * S), aux.reshape(B * 128))
    new_key = outk.reshape(B, H, K_KEEP, D)
    new_value = outv.reshape(B, H, K_KEEP, D)
    return attn_out, (new_key, new_value)


# speedup vs baseline: 4.1586x; 1.0016x over previous
import functools
import math

import jax
import jax.numpy as jnp
from jax import lax
from jax.experimental import pallas as pl
from jax.experimental.pallas import tpu as pltpu
from jax.experimental.pallas import tpu_sc as plsc

B, H, Q, S, D = 8, 16, 8, 4096, 128
K_KEEP = 2048
SINK = 4
K_CAND = K_KEEP - SINK


def _attn_body(q_ref, k_ref, v_ref, o_ref, hs_ref, acc_ref):
    h = pl.program_id(1)
    q = q_ref[0, 0]
    k = k_ref[0, 0]
    v = v_ref[0, 0]
    scale = 1.0 / math.sqrt(D)
    s = jnp.dot(q, k.T, preferred_element_type=jnp.float32) * scale
    m = jnp.max(s, axis=-1, keepdims=True)
    p = jnp.exp(s - m)
    l = jnp.sum(p, axis=-1, keepdims=True)
    w = p / l
    o_ref[0, 0] = jnp.dot(w, v, preferred_element_type=jnp.float32)
    wsum = jnp.sum(w, axis=0, keepdims=True)
    acc_ref[pl.ds(h, 1), :] = wsum

    @pl.when(h == H - 1)
    def _():
        a = acc_ref[...]
        t = a[0:8] + a[8:16]
        t = t[0:4] + t[4:8]
        t = t[0:2] + t[2:4]
        hs_ref[0] = t[0:1] + t[1:2]


def _tc_attention(query, key, value):
    grid = (B, H)
    out = pl.pallas_call(
        _attn_body,
        grid=grid,
        in_specs=[
            pl.BlockSpec((1, 1, Q, D), lambda b, h: (b, h, 0, 0)),
            pl.BlockSpec((1, 1, S, D), lambda b, h: (b, h, 0, 0)),
            pl.BlockSpec((1, 1, S, D), lambda b, h: (b, h, 0, 0)),
        ],
        out_specs=[
            pl.BlockSpec((1, 1, Q, D), lambda b, h: (b, h, 0, 0)),
            pl.BlockSpec((1, 1, S), lambda b, h: (b, 0, 0)),
        ],
        out_shape=[
            jax.ShapeDtypeStruct((B, H, Q, D), jnp.float32),
            jax.ShapeDtypeStruct((B, 1, S), jnp.float32),
        ],
        scratch_shapes=[pltpu.VMEM((H, S), jnp.float32)],
    )(query, key, value)
    return out


def _select_body(hs_ref, cls_ref, aux_ref):
    hv = hs_ref[...]
    bits = lax.bitcast_convert_type(hv, jnp.int32)
    pos = (lax.broadcasted_iota(jnp.int32, hv.shape, 1) * 128
           + lax.broadcasted_iota(jnp.int32, hv.shape, 2))
    iscand = pos >= SINK

    def bs_body(_, lohi):
        lo, hi = lohi
        mid = lo + (hi - lo + 1) // 2
        cnt = jnp.sum(jnp.where(iscand & (bits >= mid), 1, 0),
                      axis=(1, 2), keepdims=True)
        take = cnt >= K_CAND
        return (jnp.where(take, mid, lo), jnp.where(take, hi, mid - 1))

    init = (jnp.zeros((B, 1, 1), jnp.int32),
            jnp.full((B, 1, 1), 2**31 - 2, jnp.int32))
    lo, _ = lax.fori_loop(0, 31, bs_body, init)
    t = lo
    gt = iscand & (bits > t)
    eq = iscand & (bits == t)
    g = jnp.sum(jnp.where(gt, 1, 0), axis=(1, 2), keepdims=True)
    r = K_CAND - g
    cls_ref[...] = jnp.where(pos < SINK, 2,
                             jnp.where(gt, 2, jnp.where(eq, 1, 0)))
    aux_ref[...] = jnp.broadcast_to(r, (B, 1, 128)).astype(jnp.int32)


def _tc_select(head_sum):
    SCH = S // 128
    out = pl.pallas_call(
        _select_body,
        out_shape=[
            jax.ShapeDtypeStruct((B, SCH, 128), jnp.int32),
            jax.ShapeDtypeStruct((B, 1, 128), jnp.int32),
        ],
    )(head_sum.reshape(B, SCH, 128))
    return out


NC, NS, L = 2, 16, 16
NW = NC * NS
PAIRS = (B * H) // NW
CHUNK = 128
NCHUNK = K_KEEP // CHUNK


def _sc_evict(key_flat, value_flat, cls, aux):
    mesh = plsc.VectorSubcoreMesh(core_axis_name="c", subcore_axis_name="s")

    @functools.partial(
        pl.kernel,
        mesh=mesh,
        compiler_params=pltpu.CompilerParams(needs_layout_passes=False),
        out_type=[
            jax.ShapeDtypeStruct((B * H * K_KEEP, D), jnp.float32),
            jax.ShapeDtypeStruct((B * H * K_KEEP, D), jnp.float32),
        ],
        scratch_types=[
            pltpu.VMEM((S,), jnp.int32),
            pltpu.VMEM((128,), jnp.int32),
            pltpu.VMEM((K_KEEP + L,), jnp.int32),
            pltpu.VMEM((NCHUNK, CHUNK), jnp.int32),
            pltpu.VMEM((CHUNK, D), jnp.float32),
            pltpu.VMEM((CHUNK, D), jnp.float32),
            pltpu.VMEM((CHUNK, D), jnp.float32),
            pltpu.VMEM((CHUNK, D), jnp.float32),
            pltpu.SemaphoreType.DMA,
            pltpu.SemaphoreType.DMA,
            pltpu.SemaphoreType.DMA,
            pltpu.SemaphoreType.DMA,
            pltpu.SemaphoreType.DMA,
            pltpu.SemaphoreType.DMA,
            pltpu.SemaphoreType.DMA,
            pltpu.SemaphoreType.DMA,
        ],
    )
    def body(key_hbm, value_hbm, cls_hbm, aux_hbm, outk_hbm, outv_hbm,
             cls_v, aux_v, idx_v, rows_v, bk0, bk1, bv0, bv1,
             gsk0, gsk1, gsv0, gsv1, wsk0, wsk1, wsv0, wsv1):
        cid = lax.axis_index("c")
        sid = lax.axis_index("s")
        wid = sid * NC + cid
        b = wid % B
        hgrp = wid // B

        pltpu.sync_copy(cls_hbm.at[pl.ds(pl.multiple_of(b * S, S), S)], cls_v)
        pltpu.sync_copy(aux_hbm.at[pl.ds(pl.multiple_of(b * 128, 128), 128)],
                        aux_v)
        r = aux_v[pl.ds(0, L)][0]

        def comp_body(i, carry):
            nw_, nt_ = carry
            v = cls_v[pl.ds(i * L, L)]
            posv = i * L + lax.iota(jnp.int32, L)
            is2 = v == 2
            is1 = v == 1
            tp = plsc.cumsum(jnp.where(is1, 1, 0))
            take1 = is1 & ((nt_ + tp) <= r)
            keep = jnp.logical_or(is2, take1)
            plsc.store_compressed(idx_v.at[pl.ds(nw_, L)], posv, mask=keep)
            nk = plsc.all_reduce_population_count(keep)[0]
            ntk = plsc.all_reduce_population_count(take1)[0]
            return (nw_ + nk, nt_ + ntk)

        lax.fori_loop(0, S // L, comp_body, (jnp.int32(0), jnp.int32(0)))

        def pair_body(j, _):
            h = hgrp * PAIRS + j
            tbl_off = (b * H + h) * S
            out_base = (b * H + h) * K_KEEP

            def rows_body(i, _):
                c = i // (CHUNK // L)
                o = (i % (CHUNK // L)) * L
                rows_v[c, pl.ds(o, L)] = idx_v[pl.ds(i * L, L)] + tbl_off
                return 0

            lax.fori_loop(0, K_KEEP // L, rows_body, 0)

            bufs = ((bk0, bv0, gsk0, gsv0, wsk0, wsv0),
                    (bk1, bv1, gsk1, gsv1, wsk1, wsv1))

            def _dst(c):
                return pl.ds(pl.multiple_of(out_base + c * CHUNK, CHUNK),
                             CHUNK)

            def _gather(c, bk, bv, gsk, gsv):
                pltpu.async_copy(key_hbm.at[rows_v.at[c]], bk, gsk)
                pltpu.async_copy(value_hbm.at[rows_v.at[c]], bv, gsv)

            def _wait(src, dstref, sem):
                pltpu.make_async_copy(src, dstref, sem).wait()

            def step(i, bk, bv, gsk, gsv, wsk, wsv,
                     bkq, bvq, gskq, gsvq, wskq, wsvq):
                @pl.when(i < NCHUNK)
                def _():
                    @pl.when(i >= 2)
                    def _():
                        _wait(bk, outk_hbm.at[_dst(i - 2)], wsk)
                        _wait(bv, outv_hbm.at[_dst(i - 2)], wsv)
                    _gather(i, bk, bv, gsk, gsv)

                j = i - 1

                @pl.when(jnp.logical_and(j >= 0, j < NCHUNK))
                def _():
                    _wait(key_hbm.at[pl.ds(0, CHUNK)], bkq, gskq)
                    _wait(value_hbm.at[pl.ds(0, CHUNK)], bvq, gsvq)
                    pltpu.async_copy(bkq, outk_hbm.at[_dst(j)], wskq)
                    pltpu.async_copy(bvq, outv_hbm.at[_dst(j)], wsvq)

            def pipe_body(g, _):
                i0 = g * 2
                step(i0, *bufs[0], *bufs[1])
                step(i0 + 1, *bufs[1], *bufs[0])
                return 0

            lax.fori_loop(0, (NCHUNK + 2 + 1) // 2, pipe_body, 0)
            _wait(bk0, outk_hbm.at[_dst(NCHUNK - 2)], wsk0)
            _wait(bv0, outv_hbm.at[_dst(NCHUNK - 2)], wsv0)
            _wait(bk1, outk_hbm.at[_dst(NCHUNK - 1)], wsk1)
            _wait(bv1, outv_hbm.at[_dst(NCHUNK - 1)], wsv1)
            return 0

        lax.fori_loop(0, PAIRS, pair_body, 0)

    return body(key_flat, value_flat, cls, aux)


def kernel(query, key, value):
    attn_out, head_sum = _tc_attention(query, key, value)
    cls, aux = _tc_select(head_sum)
    key_flat = key.reshape(B * H * S, D)
    value_flat = value.reshape(B * H * S, D)
    outk, outv = _sc_evict(key_flat, value_flat,
                           cls.reshape(B * S), aux.reshape(B * 128))
    new_key = outk.reshape(B, H, K_KEEP, D)
    new_value = outv.reshape(B, H, K_KEEP, D)
    return attn_out, (new_key, new_value)


# speedup vs baseline: 4.5249x; 1.0881x over previous
import functools
import math

import jax
import jax.numpy as jnp
from jax import lax
from jax.experimental import pallas as pl
from jax.experimental.pallas import tpu as pltpu
from jax.experimental.pallas import tpu_sc as plsc

B, H, Q, S, D = 8, 16, 8, 4096, 128
K_KEEP = 2048
SINK = 4
K_CAND = K_KEEP - SINK


HPB = 2


def _attn_body(q_ref, k_ref, v_ref, o_ref, hs_ref, acc_ref):
    hg = pl.program_id(1)
    scale = 1.0 / math.sqrt(D)
    for u in range(HPB):
        q = q_ref[0, u]
        k = k_ref[0, u]
        v = v_ref[0, u]
        s = jnp.dot(q, k.T, preferred_element_type=jnp.float32) * scale
        m = jnp.max(s, axis=-1, keepdims=True)
        p = jnp.exp(s - m)
        l = jnp.sum(p, axis=-1, keepdims=True)
        w = p / l
        o_ref[0, u] = jnp.dot(w, v, preferred_element_type=jnp.float32)
        wsum = jnp.sum(w, axis=0, keepdims=True)
        acc_ref[pl.ds(hg * HPB + u, 1), :] = wsum

    @pl.when(hg == H // HPB - 1)
    def _():
        a = acc_ref[...]
        t = a[0:8] + a[8:16]
        t = t[0:4] + t[4:8]
        t = t[0:2] + t[2:4]
        hs_ref[0] = t[0:1] + t[1:2]


def _tc_attention(query, key, value):
    grid = (B, H // HPB)
    out = pl.pallas_call(
        _attn_body,
        grid=grid,
        in_specs=[
            pl.BlockSpec((1, HPB, Q, D), lambda b, h: (b, h, 0, 0)),
            pl.BlockSpec((1, HPB, S, D), lambda b, h: (b, h, 0, 0)),
            pl.BlockSpec((1, HPB, S, D), lambda b, h: (b, h, 0, 0)),
        ],
        out_specs=[
            pl.BlockSpec((1, HPB, Q, D), lambda b, h: (b, h, 0, 0)),
            pl.BlockSpec((1, 1, S), lambda b, h: (b, 0, 0)),
        ],
        out_shape=[
            jax.ShapeDtypeStruct((B, H, Q, D), jnp.float32),
            jax.ShapeDtypeStruct((B, 1, S), jnp.float32),
        ],
        scratch_shapes=[pltpu.VMEM((H, S), jnp.float32)],
    )(query, key, value)
    return out


def _select_body(hs_ref, cls_ref, aux_ref):
    hv = hs_ref[...]
    bits = lax.bitcast_convert_type(hv, jnp.int32)
    pos = (lax.broadcasted_iota(jnp.int32, hv.shape, 1) * 128
           + lax.broadcasted_iota(jnp.int32, hv.shape, 2))
    iscand = pos >= SINK

    def bs_body(_, lohi):
        lo, hi = lohi
        mid = lo + (hi - lo + 1) // 2
        cnt = jnp.sum(jnp.where(iscand & (bits >= mid), 1, 0),
                      axis=(1, 2), keepdims=True)
        take = cnt >= K_CAND
        return (jnp.where(take, mid, lo), jnp.where(take, hi, mid - 1))

    init = (jnp.zeros((B, 1, 1), jnp.int32),
            jnp.full((B, 1, 1), 2**31 - 2, jnp.int32))
    lo, _ = lax.fori_loop(0, 31, bs_body, init)
    t = lo
    gt = iscand & (bits > t)
    eq = iscand & (bits == t)
    g = jnp.sum(jnp.where(gt, 1, 0), axis=(1, 2), keepdims=True)
    r = K_CAND - g
    cls_ref[...] = jnp.where(pos < SINK, 2,
                             jnp.where(gt, 2, jnp.where(eq, 1, 0)))
    aux_ref[...] = jnp.broadcast_to(r, (B, 1, 128)).astype(jnp.int32)


def _tc_select(head_sum):
    SCH = S // 128
    out = pl.pallas_call(
        _select_body,
        out_shape=[
            jax.ShapeDtypeStruct((B, SCH, 128), jnp.int32),
            jax.ShapeDtypeStruct((B, 1, 128), jnp.int32),
        ],
    )(head_sum.reshape(B, SCH, 128))
    return out


NC, NS, L = 2, 16, 16
NW = NC * NS
PAIRS = (B * H) // NW
CHUNK = 128
NCHUNK = K_KEEP // CHUNK


def _sc_evict(key_flat, value_flat, cls, aux):
    mesh = plsc.VectorSubcoreMesh(core_axis_name="c", subcore_axis_name="s")

    @functools.partial(
        pl.kernel,
        mesh=mesh,
        compiler_params=pltpu.CompilerParams(needs_layout_passes=False),
        out_type=[
            jax.ShapeDtypeStruct((B * H * K_KEEP, D), jnp.float32),
            jax.ShapeDtypeStruct((B * H * K_KEEP, D), jnp.float32),
        ],
        scratch_types=[
            pltpu.VMEM((S,), jnp.int32),
            pltpu.VMEM((128,), jnp.int32),
            pltpu.VMEM((K_KEEP + L,), jnp.int32),
            pltpu.VMEM((NCHUNK, CHUNK), jnp.int32),
            pltpu.VMEM((CHUNK, D), jnp.float32),
            pltpu.VMEM((CHUNK, D), jnp.float32),
            pltpu.VMEM((CHUNK, D), jnp.float32),
            pltpu.VMEM((CHUNK, D), jnp.float32),
            pltpu.SemaphoreType.DMA,
            pltpu.SemaphoreType.DMA,
            pltpu.SemaphoreType.DMA,
            pltpu.SemaphoreType.DMA,
            pltpu.SemaphoreType.DMA,
            pltpu.SemaphoreType.DMA,
            pltpu.SemaphoreType.DMA,
            pltpu.SemaphoreType.DMA,
        ],
    )
    def body(key_hbm, value_hbm, cls_hbm, aux_hbm, outk_hbm, outv_hbm,
             cls_v, aux_v, idx_v, rows_v, bk0, bk1, bv0, bv1,
             gsk0, gsk1, gsv0, gsv1, wsk0, wsk1, wsv0, wsv1):
        cid = lax.axis_index("c")
        sid = lax.axis_index("s")
        wid = sid * NC + cid
        b = wid % B
        hgrp = wid // B

        pltpu.sync_copy(cls_hbm.at[pl.ds(pl.multiple_of(b * S, S), S)], cls_v)
        pltpu.sync_copy(aux_hbm.at[pl.ds(pl.multiple_of(b * 128, 128), 128)],
                        aux_v)
        r = aux_v[pl.ds(0, L)][0]

        def comp_body(i, carry):
            nw_, nt_ = carry
            v = cls_v[pl.ds(i * L, L)]
            posv = i * L + lax.iota(jnp.int32, L)
            is2 = v == 2
            is1 = v == 1
            tp = plsc.cumsum(jnp.where(is1, 1, 0))
            take1 = is1 & ((nt_ + tp) <= r)
            keep = jnp.logical_or(is2, take1)
            plsc.store_compressed(idx_v.at[pl.ds(nw_, L)], posv, mask=keep)
            nk = plsc.all_reduce_population_count(keep)[0]
            ntk = plsc.all_reduce_population_count(take1)[0]
            return (nw_ + nk, nt_ + ntk)

        lax.fori_loop(0, S // L, comp_body, (jnp.int32(0), jnp.int32(0)))

        def pair_body(j, _):
            h = hgrp * PAIRS + j
            tbl_off = (b * H + h) * S
            out_base = (b * H + h) * K_KEEP

            def rows_body(i, _):
                c = i // (CHUNK // L)
                o = (i % (CHUNK // L)) * L
                rows_v[c, pl.ds(o, L)] = idx_v[pl.ds(i * L, L)] + tbl_off
                return 0

            lax.fori_loop(0, K_KEEP // L, rows_body, 0)

            bufs = ((bk0, bv0, gsk0, gsv0, wsk0, wsv0),
                    (bk1, bv1, gsk1, gsv1, wsk1, wsv1))

            def _dst(c):
                return pl.ds(pl.multiple_of(out_base + c * CHUNK, CHUNK),
                             CHUNK)

            def _gather(c, bk, bv, gsk, gsv):
                pltpu.async_copy(key_hbm.at[rows_v.at[c]], bk, gsk)
                pltpu.async_copy(value_hbm.at[rows_v.at[c]], bv, gsv)

            def _wait(src, dstref, sem):
                pltpu.make_async_copy(src, dstref, sem).wait()

            def step(i, bk, bv, gsk, gsv, wsk, wsv,
                     bkq, bvq, gskq, gsvq, wskq, wsvq):
                @pl.when(i < NCHUNK)
                def _():
                    @pl.when(i >= 2)
                    def _():
                        _wait(bk, outk_hbm.at[_dst(i - 2)], wsk)
                        _wait(bv, outv_hbm.at[_dst(i - 2)], wsv)
                    _gather(i, bk, bv, gsk, gsv)

                j = i - 1

                @pl.when(jnp.logical_and(j >= 0, j < NCHUNK))
                def _():
                    _wait(key_hbm.at[pl.ds(0, CHUNK)], bkq, gskq)
                    _wait(value_hbm.at[pl.ds(0, CHUNK)], bvq, gsvq)
                    pltpu.async_copy(bkq, outk_hbm.at[_dst(j)], wskq)
                    pltpu.async_copy(bvq, outv_hbm.at[_dst(j)], wsvq)

            def pipe_body(g, _):
                i0 = g * 2
                step(i0, *bufs[0], *bufs[1])
                step(i0 + 1, *bufs[1], *bufs[0])
                return 0

            lax.fori_loop(0, (NCHUNK + 2 + 1) // 2, pipe_body, 0)
            _wait(bk0, outk_hbm.at[_dst(NCHUNK - 2)], wsk0)
            _wait(bv0, outv_hbm.at[_dst(NCHUNK - 2)], wsv0)
            _wait(bk1, outk_hbm.at[_dst(NCHUNK - 1)], wsk1)
            _wait(bv1, outv_hbm.at[_dst(NCHUNK - 1)], wsv1)
            return 0

        lax.fori_loop(0, PAIRS, pair_body, 0)

    return body(key_flat, value_flat, cls, aux)


def kernel(query, key, value):
    attn_out, head_sum = _tc_attention(query, key, value)
    cls, aux = _tc_select(head_sum)
    key_flat = key.reshape(B * H * S, D)
    value_flat = value.reshape(B * H * S, D)
    outk, outv = _sc_evict(key_flat, value_flat,
                           cls.reshape(B * S), aux.reshape(B * 128))
    new_key = outk.reshape(B, H, K_KEEP, D)
    new_value = outv.reshape(B, H, K_KEEP, D)
    return attn_out, (new_key, new_value)


# speedup vs baseline: 4.6733x; 1.0328x over previous
import functools
import math

import jax
import jax.numpy as jnp
from jax import lax
from jax.experimental import pallas as pl
from jax.experimental.pallas import tpu as pltpu
from jax.experimental.pallas import tpu_sc as plsc

B, H, Q, S, D = 8, 16, 8, 4096, 128
K_KEEP = 2048
SINK = 4
K_CAND = K_KEEP - SINK


HPB = 4


def _attn_body(q_ref, k_ref, v_ref, o_ref, hs_ref, acc_ref):
    hg = pl.program_id(1)
    scale = 1.0 / math.sqrt(D)
    for u in range(HPB):
        q = q_ref[0, u]
        k = k_ref[0, u]
        v = v_ref[0, u]
        s = jnp.dot(q, k.T, preferred_element_type=jnp.float32) * scale
        m = jnp.max(s, axis=-1, keepdims=True)
        p = jnp.exp(s - m)
        l = jnp.sum(p, axis=-1, keepdims=True)
        w = p / l
        o_ref[0, u] = jnp.dot(w, v, preferred_element_type=jnp.float32)
        wsum = jnp.sum(w, axis=0, keepdims=True)
        acc_ref[pl.ds(hg * HPB + u, 1), :] = wsum

    @pl.when(hg == H // HPB - 1)
    def _():
        a = acc_ref[...]
        t = a[0:8] + a[8:16]
        t = t[0:4] + t[4:8]
        t = t[0:2] + t[2:4]
        hs_ref[0] = t[0:1] + t[1:2]


def _tc_attention(query, key, value):
    grid = (B, H // HPB)
    out = pl.pallas_call(
        _attn_body,
        grid=grid,
        in_specs=[
            pl.BlockSpec((1, HPB, Q, D), lambda b, h: (b, h, 0, 0)),
            pl.BlockSpec((1, HPB, S, D), lambda b, h: (b, h, 0, 0)),
            pl.BlockSpec((1, HPB, S, D), lambda b, h: (b, h, 0, 0)),
        ],
        out_specs=[
            pl.BlockSpec((1, HPB, Q, D), lambda b, h: (b, h, 0, 0)),
            pl.BlockSpec((1, 1, S), lambda b, h: (b, 0, 0)),
        ],
        out_shape=[
            jax.ShapeDtypeStruct((B, H, Q, D), jnp.float32),
            jax.ShapeDtypeStruct((B, 1, S), jnp.float32),
        ],
        scratch_shapes=[pltpu.VMEM((H, S), jnp.float32)],
    )(query, key, value)
    return out


def _select_body(hs_ref, cls_ref, aux_ref):
    hv = hs_ref[...]
    bits = lax.bitcast_convert_type(hv, jnp.int32)
    pos = (lax.broadcasted_iota(jnp.int32, hv.shape, 1) * 128
           + lax.broadcasted_iota(jnp.int32, hv.shape, 2))
    iscand = pos >= SINK

    def bs_body(_, lohi):
        lo, hi = lohi
        mid = lo + (hi - lo + 1) // 2
        cnt = jnp.sum(jnp.where(iscand & (bits >= mid), 1, 0),
                      axis=(1, 2), keepdims=True)
        take = cnt >= K_CAND
        return (jnp.where(take, mid, lo), jnp.where(take, hi, mid - 1))

    init = (jnp.zeros((B, 1, 1), jnp.int32),
            jnp.full((B, 1, 1), 2**31 - 2, jnp.int32))
    lo, _ = lax.fori_loop(0, 31, bs_body, init)
    t = lo
    gt = iscand & (bits > t)
    eq = iscand & (bits == t)
    g = jnp.sum(jnp.where(gt, 1, 0), axis=(1, 2), keepdims=True)
    r = K_CAND - g
    cls_ref[...] = jnp.where(pos < SINK, 2,
                             jnp.where(gt, 2, jnp.where(eq, 1, 0)))
    aux_ref[...] = jnp.broadcast_to(r, (B, 1, 128)).astype(jnp.int32)


def _tc_select(head_sum):
    SCH = S // 128
    out = pl.pallas_call(
        _select_body,
        out_shape=[
            jax.ShapeDtypeStruct((B, SCH, 128), jnp.int32),
            jax.ShapeDtypeStruct((B, 1, 128), jnp.int32),
        ],
    )(head_sum.reshape(B, SCH, 128))
    return out


NC, NS, L = 2, 16, 16
NW = NC * NS
PAIRS = (B * H) // NW
CHUNK = 128
NCHUNK = K_KEEP // CHUNK


def _sc_evict(key_flat, value_flat, cls, aux):
    mesh = plsc.VectorSubcoreMesh(core_axis_name="c", subcore_axis_name="s")

    @functools.partial(
        pl.kernel,
        mesh=mesh,
        compiler_params=pltpu.CompilerParams(needs_layout_passes=False),
        out_type=[
            jax.ShapeDtypeStruct((B * H * K_KEEP, D), jnp.float32),
            jax.ShapeDtypeStruct((B * H * K_KEEP, D), jnp.float32),
        ],
        scratch_types=[
            pltpu.VMEM((S,), jnp.int32),
            pltpu.VMEM((128,), jnp.int32),
            pltpu.VMEM((K_KEEP + L,), jnp.int32),
            pltpu.VMEM((NCHUNK, CHUNK), jnp.int32),
            pltpu.VMEM((CHUNK, D), jnp.float32),
            pltpu.VMEM((CHUNK, D), jnp.float32),
            pltpu.VMEM((CHUNK, D), jnp.float32),
            pltpu.VMEM((CHUNK, D), jnp.float32),
            pltpu.SemaphoreType.DMA,
            pltpu.SemaphoreType.DMA,
            pltpu.SemaphoreType.DMA,
            pltpu.SemaphoreType.DMA,
            pltpu.SemaphoreType.DMA,
            pltpu.SemaphoreType.DMA,
            pltpu.SemaphoreType.DMA,
            pltpu.SemaphoreType.DMA,
        ],
    )
    def body(key_hbm, value_hbm, cls_hbm, aux_hbm, outk_hbm, outv_hbm,
             cls_v, aux_v, idx_v, rows_v, bk0, bk1, bv0, bv1,
             gsk0, gsk1, gsv0, gsv1, wsk0, wsk1, wsv0, wsv1):
        cid = lax.axis_index("c")
        sid = lax.axis_index("s")
        wid = sid * NC + cid
        b = wid % B
        hgrp = wid // B

        pltpu.sync_copy(cls_hbm.at[pl.ds(pl.multiple_of(b * S, S), S)], cls_v)
        pltpu.sync_copy(aux_hbm.at[pl.ds(pl.multiple_of(b * 128, 128), 128)],
                        aux_v)
        r = aux_v[pl.ds(0, L)][0]

        def comp_body(i, carry):
            nw_, nt_ = carry
            v = cls_v[pl.ds(i * L, L)]
            posv = i * L + lax.iota(jnp.int32, L)
            is2 = v == 2
            is1 = v == 1
            tp = plsc.cumsum(jnp.where(is1, 1, 0))
            take1 = is1 & ((nt_ + tp) <= r)
            keep = jnp.logical_or(is2, take1)
            plsc.store_compressed(idx_v.at[pl.ds(nw_, L)], posv, mask=keep)
            nk = plsc.all_reduce_population_count(keep)[0]
            ntk = plsc.all_reduce_population_count(take1)[0]
            return (nw_ + nk, nt_ + ntk)

        lax.fori_loop(0, S // L, comp_body, (jnp.int32(0), jnp.int32(0)))

        def pair_body(j, _):
            h = hgrp * PAIRS + j
            tbl_off = (b * H + h) * S
            out_base = (b * H + h) * K_KEEP

            def rows_body(i, _):
                c = i // (CHUNK // L)
                o = (i % (CHUNK // L)) * L
                rows_v[c, pl.ds(o, L)] = idx_v[pl.ds(i * L, L)] + tbl_off
                return 0

            lax.fori_loop(0, K_KEEP // L, rows_body, 0)

            bufs = ((bk0, bv0, gsk0, gsv0, wsk0, wsv0),
                    (bk1, bv1, gsk1, gsv1, wsk1, wsv1))

            def _dst(c):
                return pl.ds(pl.multiple_of(out_base + c * CHUNK, CHUNK),
                             CHUNK)

            def _gather(c, bk, bv, gsk, gsv):
                pltpu.async_copy(key_hbm.at[rows_v.at[c]], bk, gsk)
                pltpu.async_copy(value_hbm.at[rows_v.at[c]], bv, gsv)

            def _wait(src, dstref, sem):
                pltpu.make_async_copy(src, dstref, sem).wait()

            def step(i, bk, bv, gsk, gsv, wsk, wsv,
                     bkq, bvq, gskq, gsvq, wskq, wsvq):
                @pl.when(i < NCHUNK)
                def _():
                    @pl.when(i >= 2)
                    def _():
                        _wait(bk, outk_hbm.at[_dst(i - 2)], wsk)
                        _wait(bv, outv_hbm.at[_dst(i - 2)], wsv)
                    _gather(i, bk, bv, gsk, gsv)

                j = i - 1

                @pl.when(jnp.logical_and(j >= 0, j < NCHUNK))
                def _():
                    _wait(key_hbm.at[pl.ds(0, CHUNK)], bkq, gskq)
                    _wait(value_hbm.at[pl.ds(0, CHUNK)], bvq, gsvq)
                    pltpu.async_copy(bkq, outk_hbm.at[_dst(j)], wskq)
                    pltpu.async_copy(bvq, outv_hbm.at[_dst(j)], wsvq)

            def pipe_body(g, _):
                i0 = g * 2
                step(i0, *bufs[0], *bufs[1])
                step(i0 + 1, *bufs[1], *bufs[0])
                return 0

            lax.fori_loop(0, (NCHUNK + 2 + 1) // 2, pipe_body, 0)
            _wait(bk0, outk_hbm.at[_dst(NCHUNK - 2)], wsk0)
            _wait(bv0, outv_hbm.at[_dst(NCHUNK - 2)], wsv0)
            _wait(bk1, outk_hbm.at[_dst(NCHUNK - 1)], wsk1)
            _wait(bv1, outv_hbm.at[_dst(NCHUNK - 1)], wsv1)
            return 0

        lax.fori_loop(0, PAIRS, pair_body, 0)

    return body(key_flat, value_flat, cls, aux)


def kernel(query, key, value):
    attn_out, head_sum = _tc_attention(query, key, value)
    cls, aux = _tc_select(head_sum)
    key_flat = key.reshape(B * H * S, D)
    value_flat = value.reshape(B * H * S, D)
    outk, outv = _sc_evict(key_flat, value_flat,
                           cls.reshape(B * S), aux.reshape(B * 128))
    new_key = outk.reshape(B, H, K_KEEP, D)
    new_value = outv.reshape(B, H, K_KEEP, D)
    return attn_out, (new_key, new_value)


# speedup vs baseline: 4.6987x; 1.0054x over previous
import functools
import math

import jax
import jax.numpy as jnp
from jax import lax
from jax.experimental import pallas as pl
from jax.experimental.pallas import tpu as pltpu
from jax.experimental.pallas import tpu_sc as plsc

B, H, Q, S, D = 8, 16, 8, 4096, 128
K_KEEP = 2048
SINK = 4
K_CAND = K_KEEP - SINK


HPB = 4


def _attn_body(q_ref, k_ref, v_ref, o_ref, hs_ref, acc_ref):
    hg = pl.program_id(1)
    scale = 1.0 / math.sqrt(D)
    for u in range(HPB):
        q = q_ref[0, u]
        k = k_ref[0, u]
        v = v_ref[0, u]
        s = jnp.dot(q, k.T, preferred_element_type=jnp.float32) * scale
        m = jnp.max(s, axis=-1, keepdims=True)
        p = jnp.exp(s - m)
        l = jnp.sum(p, axis=-1, keepdims=True)
        w = p / l
        o_ref[0, u] = jnp.dot(w, v, preferred_element_type=jnp.float32)
        wsum = jnp.sum(w, axis=0, keepdims=True)
        acc_ref[pl.ds(hg * HPB + u, 1), :] = wsum

    @pl.when(hg == H // HPB - 1)
    def _():
        a = acc_ref[...]
        t = a[0:8] + a[8:16]
        t = t[0:4] + t[4:8]
        t = t[0:2] + t[2:4]
        hs_ref[0] = t[0:1] + t[1:2]


def _tc_attention(query, key, value):
    grid = (B, H // HPB)
    out = pl.pallas_call(
        _attn_body,
        grid=grid,
        in_specs=[
            pl.BlockSpec((1, HPB, Q, D), lambda b, h: (b, h, 0, 0)),
            pl.BlockSpec((1, HPB, S, D), lambda b, h: (b, h, 0, 0)),
            pl.BlockSpec((1, HPB, S, D), lambda b, h: (b, h, 0, 0)),
        ],
        out_specs=[
            pl.BlockSpec((1, HPB, Q, D), lambda b, h: (b, h, 0, 0)),
            pl.BlockSpec((1, 1, S), lambda b, h: (b, 0, 0)),
        ],
        out_shape=[
            jax.ShapeDtypeStruct((B, H, Q, D), jnp.float32),
            jax.ShapeDtypeStruct((B, 1, S), jnp.float32),
        ],
        scratch_shapes=[pltpu.VMEM((H, S), jnp.float32)],
    )(query, key, value)
    return out


def _select_body(hs_ref, cls_ref, aux_ref):
    hv = hs_ref[...]
    bits = lax.bitcast_convert_type(hv, jnp.int32)
    pos = (lax.broadcasted_iota(jnp.int32, hv.shape, 1) * 128
           + lax.broadcasted_iota(jnp.int32, hv.shape, 2))
    iscand = pos >= SINK

    def bs_body(_, lohi):
        lo, hi = lohi
        mid = lo + (hi - lo + 1) // 2
        cnt = jnp.sum(jnp.where(iscand & (bits >= mid), 1, 0),
                      axis=(1, 2), keepdims=True)
        take = cnt >= K_CAND
        return (jnp.where(take, mid, lo), jnp.where(take, hi, mid - 1))

    init = (jnp.zeros((B, 1, 1), jnp.int32),
            jnp.full((B, 1, 1), 2**31 - 2, jnp.int32))
    lo, _ = lax.fori_loop(0, 31, bs_body, init)
    t = lo
    gt = iscand & (bits > t)
    eq = iscand & (bits == t)
    g = jnp.sum(jnp.where(gt, 1, 0), axis=(1, 2), keepdims=True)
    r = K_CAND - g
    cls_ref[...] = jnp.where(pos < SINK, 2,
                             jnp.where(gt, 2, jnp.where(eq, 1, 0)))
    aux_ref[...] = jnp.broadcast_to(r, (B, 1, 128)).astype(jnp.int32)


def _tc_select(head_sum):
    SCH = S // 128
    out = pl.pallas_call(
        _select_body,
        out_shape=[
            jax.ShapeDtypeStruct((B, SCH, 128), jnp.int32),
            jax.ShapeDtypeStruct((B, 1, 128), jnp.int32),
        ],
    )(head_sum.reshape(B, SCH, 128))
    return out


NC, NS, L = 2, 16, 16
NW = NC * NS
PAIRS = (B * H) // NW
CHUNK = 128
NCHUNK = K_KEEP // CHUNK
NBUF = 3
LOOKA = 2


def _sc_evict(key_flat, value_flat, cls, aux):
    mesh = plsc.VectorSubcoreMesh(core_axis_name="c", subcore_axis_name="s")

    @functools.partial(
        pl.kernel,
        mesh=mesh,
        compiler_params=pltpu.CompilerParams(needs_layout_passes=False),
        out_type=[
            jax.ShapeDtypeStruct((B * H * K_KEEP, D), jnp.float32),
            jax.ShapeDtypeStruct((B * H * K_KEEP, D), jnp.float32),
        ],
        scratch_types=[
            pltpu.VMEM((S,), jnp.int32),
            pltpu.VMEM((128,), jnp.int32),
            pltpu.VMEM((K_KEEP + L,), jnp.int32),
            pltpu.VMEM((NCHUNK, CHUNK), jnp.int32),
        ] + [pltpu.VMEM((CHUNK, D), jnp.float32)] * (2 * NBUF)
          + [pltpu.SemaphoreType.DMA] * (4 * NBUF),
    )
    def body(key_hbm, value_hbm, cls_hbm, aux_hbm, outk_hbm, outv_hbm,
             cls_v, aux_v, idx_v, rows_v, *bufsem):
        bks = bufsem[0:NBUF]
        bvs = bufsem[NBUF:2 * NBUF]
        gsk = bufsem[2 * NBUF:3 * NBUF]
        gsv = bufsem[3 * NBUF:4 * NBUF]
        wsk = bufsem[4 * NBUF:5 * NBUF]
        wsv = bufsem[5 * NBUF:6 * NBUF]
        cid = lax.axis_index("c")
        sid = lax.axis_index("s")
        wid = sid * NC + cid
        b = wid % B
        hgrp = wid // B

        pltpu.sync_copy(cls_hbm.at[pl.ds(pl.multiple_of(b * S, S), S)], cls_v)
        pltpu.sync_copy(aux_hbm.at[pl.ds(pl.multiple_of(b * 128, 128), 128)],
                        aux_v)
        r = aux_v[pl.ds(0, L)][0]

        def comp_body(i, carry):
            nw_, nt_ = carry
            v = cls_v[pl.ds(i * L, L)]
            posv = i * L + lax.iota(jnp.int32, L)
            is2 = v == 2
            is1 = v == 1
            tp = plsc.cumsum(jnp.where(is1, 1, 0))
            take1 = is1 & ((nt_ + tp) <= r)
            keep = jnp.logical_or(is2, take1)
            plsc.store_compressed(idx_v.at[pl.ds(nw_, L)], posv, mask=keep)
            nk = plsc.all_reduce_population_count(keep)[0]
            ntk = plsc.all_reduce_population_count(take1)[0]
            return (nw_ + nk, nt_ + ntk)

        lax.fori_loop(0, S // L, comp_body, (jnp.int32(0), jnp.int32(0)))

        def pair_body(j, _):
            h = hgrp * PAIRS + j
            tbl_off = (b * H + h) * S
            out_base = (b * H + h) * K_KEEP

            def rows_body(i, _):
                c = i // (CHUNK // L)
                o = (i % (CHUNK // L)) * L
                rows_v[c, pl.ds(o, L)] = idx_v[pl.ds(i * L, L)] + tbl_off
                return 0

            lax.fori_loop(0, K_KEEP // L, rows_body, 0)

            def _dst(c):
                return pl.ds(pl.multiple_of(out_base + c * CHUNK, CHUNK),
                             CHUNK)

            def _wait(src, dstref, sem):
                pltpu.make_async_copy(src, dstref, sem).wait()

            def step(i, p):
                @pl.when(i < NCHUNK)
                def _():
                    @pl.when(i >= NBUF)
                    def _():
                        _wait(bks[p], outk_hbm.at[_dst(i - NBUF)], wsk[p])
                        _wait(bvs[p], outv_hbm.at[_dst(i - NBUF)], wsv[p])
                    pltpu.async_copy(key_hbm.at[rows_v.at[i]], bks[p], gsk[p])
                    pltpu.async_copy(value_hbm.at[rows_v.at[i]], bvs[p], gsv[p])

                j = i - LOOKA
                q = (p - LOOKA) % NBUF

                @pl.when(jnp.logical_and(j >= 0, j < NCHUNK))
                def _():
                    _wait(key_hbm.at[pl.ds(0, CHUNK)], bks[q], gsk[q])
                    _wait(value_hbm.at[pl.ds(0, CHUNK)], bvs[q], gsv[q])
                    pltpu.async_copy(bks[q], outk_hbm.at[_dst(j)], wsk[q])
                    pltpu.async_copy(bvs[q], outv_hbm.at[_dst(j)], wsv[q])

            def pipe_body(g, _):
                for p in range(NBUF):
                    step(g * NBUF + p, p)
                return 0

            lax.fori_loop(0, -(-(NCHUNK + LOOKA) // NBUF), pipe_body, 0)
            for c in range(NCHUNK - NBUF, NCHUNK):
                _wait(bks[c % NBUF], outk_hbm.at[_dst(c)], wsk[c % NBUF])
                _wait(bvs[c % NBUF], outv_hbm.at[_dst(c)], wsv[c % NBUF])
            return 0

        lax.fori_loop(0, PAIRS, pair_body, 0)

    return body(key_flat, value_flat, cls, aux)


def kernel(query, key, value):
    attn_out, head_sum = _tc_attention(query, key, value)
    cls, aux = _tc_select(head_sum)
    key_flat = key.reshape(B * H * S, D)
    value_flat = value.reshape(B * H * S, D)
    outk, outv = _sc_evict(key_flat, value_flat,
                           cls.reshape(B * S), aux.reshape(B * 128))
    new_key = outk.reshape(B, H, K_KEEP, D)
    new_value = outv.reshape(B, H, K_KEEP, D)
    return attn_out, (new_key, new_value)
